# baseline (speedup 1.0000x reference)
import functools

import jax
import jax.numpy as jnp
from jax import lax
from jax.experimental import pallas as pl
from jax.experimental.pallas import tpu as pltpu
from jax.experimental.pallas import tpu_sc as plsc

N = 2048
D = 1024
BLK1 = 256
BLKS = 512
NW = 32
ROWS_PER_W = N // NW
L = 16


def _stage1_body(x_ref, ws_ref, bs_ref, ww_ref, bw_ref, wv_ref, bv_ref,
                 idx_ref, w_ref, v_ref):
    x = x_ref[...]
    dims = (((1,), (1,)), ((), ()))
    ls = lax.dot_general(x, ws_ref[...], dims,
                         preferred_element_type=jnp.float32) + bs_ref[...]
    m = jnp.max(ls, axis=-1, keepdims=True)
    e = jnp.exp(ls - m)
    p = e / jnp.sum(e, axis=-1, keepdims=True)
    idx = jnp.minimum((p * N).astype(jnp.int32), N - 1)
    idx_ref[...] = jnp.bitwise_or(idx[:, : D // 2],
                                  lax.shift_left(idx[:, D // 2:], 16))

    xb = x.astype(jnp.bfloat16)
    lw = lax.dot_general(xb, ww_ref[...].astype(jnp.bfloat16), dims,
                         preferred_element_type=jnp.float32) + bw_ref[...]
    mw = jnp.max(lw, axis=-1, keepdims=True)
    ew = jnp.exp(lw - mw)
    w_ref[...] = (ew / jnp.sum(ew, axis=-1, keepdims=True)).astype(jnp.bfloat16)

    v_ref[...] = (lax.dot_general(xb, wv_ref[...].astype(jnp.bfloat16), dims,
                                  preferred_element_type=jnp.float32)
                  + bv_ref[...]).astype(jnp.bfloat16)


def _stage1(x, Ws, bs, Ww, bw, Wv, bv):
    full = pl.BlockSpec((D, D), lambda i: (0, 0))
    bias = pl.BlockSpec((1, D), lambda i: (0, 0))
    rows = pl.BlockSpec((BLK1, D), lambda i: (i, 0))
    rows_h = pl.BlockSpec((BLK1, D // 2), lambda i: (i, 0))
    return pl.pallas_call(
        _stage1_body,
        grid=(N // BLK1,),
        in_specs=[rows, full, bias, full, bias, full, bias],
        out_specs=[rows_h, rows, rows],
        out_shape=[
            jax.ShapeDtypeStruct((N, D // 2), jnp.int32),
            jax.ShapeDtypeStruct((N, D), jnp.bfloat16),
            jax.ShapeDtypeStruct((N, D), jnp.bfloat16),
        ],
    )(x, Ws, bs.reshape(1, D), Ww, bw.reshape(1, D), Wv, bv.reshape(1, D))


def _scores_body(w_ref, v_ref, s_ref):
    s = lax.dot_general(
        w_ref[...], v_ref[...], (((1,), (1,)), ((), ())),
        preferred_element_type=jnp.float32)
    lo = lax.bitcast_convert_type(s[:, : N // 2].astype(jnp.bfloat16)
                                  .astype(jnp.float32), jnp.int32)
    hi = lax.bitcast_convert_type(s[:, N // 2:].astype(jnp.bfloat16)
                                  .astype(jnp.float32), jnp.int32)
    s_ref[...] = jnp.bitwise_or(
        lax.shift_right_logical(lo, 16),
        jnp.bitwise_and(hi, jnp.int32(-65536)))


def _scores(w, values):
    return pl.pallas_call(
        _scores_body,
        grid=(N // BLKS,),
        in_specs=[
            pl.BlockSpec((BLKS, D), lambda i: (i, 0)),
            pl.BlockSpec((N, D), lambda i: (0, 0)),
        ],
        out_specs=pl.BlockSpec((BLKS, N // 2), lambda i: (i, 0)),
        out_shape=jax.ShapeDtypeStruct((N, N // 2), jnp.int32),
    )(w, values)


CH = 16
NCHUNK = ROWS_PER_W // CH


def _gather_body(scores_hbm, idx_hbm, out_hbm,
                 scr_v, idx_v, out_v, sem_in, sem_out):
    wid = lax.axis_index("s") * 2 + lax.axis_index("c")
    base = wid * ROWS_PER_W

    def start_in(c, b):
        rows = pl.ds(base + c * CH, CH)
        d1 = pltpu.async_copy(scores_hbm.at[rows], scr_v.at[b], sem_in.at[b])
        d2 = pltpu.async_copy(idx_hbm.at[rows], idx_v.at[b], sem_in.at[b])
        return d1, d2

    in_flight = {0: start_in(0, 0)}
    out_flight = {}
    for c in range(NCHUNK):
        b = c % 2
        if c + 1 < NCHUNK:
            in_flight[c + 1] = start_in(c + 1, (c + 1) % 2)
        for dsc in in_flight.pop(c):
            dsc.wait()
        if c - 2 in out_flight:
            out_flight.pop(c - 2).wait()
        for r in range(CH):
            rv = jnp.full((L,), r, jnp.int32)

            def gat(j, _, r=r, rv=rv, b=b):
                ivw = idx_v[b, r, pl.ds(j * L, L)]

                def one(iv):
                    w32 = plsc.load_gather(
                        scr_v.at[b], [rv, jnp.bitwise_and(iv, N // 2 - 1)])
                    sh = jnp.bitwise_and(lax.shift_right_logical(iv, 6), 16)
                    return lax.shift_left(lax.shift_right_logical(w32, sh), 16)

                blo = one(jnp.bitwise_and(ivw, N - 1))
                bhi = one(lax.shift_right_logical(ivw, 16))
                out_v[b, r, pl.ds(j * L, L)] = jnp.bitwise_or(
                    lax.shift_right_logical(blo, 16), bhi)
                return 0

            lax.fori_loop(0, D // (2 * L), gat, 0, unroll=4)
        out_flight[c] = pltpu.async_copy(
            out_v.at[b], out_hbm.at[pl.ds(base + c * CH, CH)], sem_out.at[b])
    for c in list(out_flight):
        out_flight.pop(c).wait()


def _sc_gather(scores, idx):
    mesh = plsc.VectorSubcoreMesh(core_axis_name="c", subcore_axis_name="s")
    f = pl.kernel(
        _gather_body,
        out_type=jax.ShapeDtypeStruct((N, D // 2), jnp.int32),
        mesh=mesh,
        compiler_params=pltpu.CompilerParams(needs_layout_passes=False),
        scratch_types=[
            pltpu.VMEM((2, CH, N // 2), jnp.int32),
            pltpu.VMEM((2, CH, D // 2), jnp.int32),
            pltpu.VMEM((2, CH, D // 2), jnp.int32),
            pltpu.SemaphoreType.DMA((2,)),
            pltpu.SemaphoreType.DMA((2,)),
        ],
    )
    return f(scores, idx)


def _out_body(vs_ref, wo_ref, bo_ref, o_ref):
    w32 = vs_ref[...]
    lo = lax.bitcast_convert_type(lax.shift_left(w32, 16), jnp.float32)
    hi = lax.bitcast_convert_type(
        jnp.bitwise_and(w32, jnp.int32(-65536)), jnp.float32)
    vs = jnp.concatenate([lo, hi], axis=1).astype(jnp.bfloat16)
    o_ref[...] = lax.dot_general(
        vs, wo_ref[...].astype(jnp.bfloat16),
        (((1,), (1,)), ((), ())),
        preferred_element_type=jnp.float32) + bo_ref[...]


def _out(vs, Wo, bo):
    return pl.pallas_call(
        _out_body,
        grid=(N // BLK1,),
        in_specs=[
            pl.BlockSpec((BLK1, D // 2), lambda i: (i, 0)),
            pl.BlockSpec((D, D), lambda i: (0, 0)),
            pl.BlockSpec((1, D), lambda i: (0, 0)),
        ],
        out_specs=pl.BlockSpec((BLK1, D), lambda i: (i, 0)),
        out_shape=jax.ShapeDtypeStruct((N, D), jnp.float32),
    )(vs, Wo, bo.reshape(1, D))


def kernel(x, Ws, bs, Ww, bw, Wv, bv, Wo, bo):
    idx, w, values = _stage1(x, Ws, bs, Ww, bw, Wv, bv)
    scores_w = _scores(w, values)
    vs = _sc_gather(scores_w, idx)
    return _out(vs, Wo, bo)


# speedup vs baseline: 1.0649x; 1.0649x over previous
import functools

import jax
import jax.numpy as jnp
from jax import lax
from jax.experimental import pallas as pl
from jax.experimental.pallas import tpu as pltpu
from jax.experimental.pallas import tpu_sc as plsc

N = 2048
D = 1024
BLK1 = 256
BLKS = 512
NW = 32
ROWS_PER_W = N // NW
L = 16


def _stage1_body(x_ref, ws_ref, bs_ref, ww_ref, bw_ref, wv_ref, bv_ref,
                 idx_ref, w_ref, v_ref):
    x = x_ref[...]
    dims = (((1,), (1,)), ((), ()))
    ls = lax.dot_general(x, ws_ref[...], dims,
                         preferred_element_type=jnp.float32) + bs_ref[...]
    m = jnp.max(ls, axis=-1, keepdims=True)
    e = jnp.exp(ls - m)
    p = e / jnp.sum(e, axis=-1, keepdims=True)
    idx = jnp.minimum((p * N).astype(jnp.int32), N - 1)
    idx_ref[...] = jnp.bitwise_or(idx[:, : D // 2],
                                  lax.shift_left(idx[:, D // 2:], 16))

    xb = x.astype(jnp.bfloat16)
    lw = lax.dot_general(xb, ww_ref[...].astype(jnp.bfloat16), dims,
                         preferred_element_type=jnp.float32) + bw_ref[...]
    mw = jnp.max(lw, axis=-1, keepdims=True)
    ew = jnp.exp(lw - mw)
    w_ref[...] = (ew / jnp.sum(ew, axis=-1, keepdims=True)).astype(jnp.bfloat16)

    v_ref[...] = (lax.dot_general(xb, wv_ref[...].astype(jnp.bfloat16), dims,
                                  preferred_element_type=jnp.float32)
                  + bv_ref[...]).astype(jnp.bfloat16)


def _stage1(x, Ws, bs, Ww, bw, Wv, bv):
    full = pl.BlockSpec((D, D), lambda i: (0, 0))
    bias = pl.BlockSpec((1, D), lambda i: (0, 0))
    rows = pl.BlockSpec((BLK1, D), lambda i: (i, 0))
    rows_h = pl.BlockSpec((BLK1, D // 2), lambda i: (i, 0))
    return pl.pallas_call(
        _stage1_body,
        grid=(N // BLK1,),
        in_specs=[rows, full, bias, full, bias, full, bias],
        out_specs=[rows_h, rows, rows],
        out_shape=[
            jax.ShapeDtypeStruct((N, D // 2), jnp.int32),
            jax.ShapeDtypeStruct((N, D), jnp.bfloat16),
            jax.ShapeDtypeStruct((N, D), jnp.bfloat16),
        ],
    )(x, Ws, bs.reshape(1, D), Ww, bw.reshape(1, D), Wv, bv.reshape(1, D))


def _scores_body(w_ref, v_ref, s_ref):
    s = lax.dot_general(
        w_ref[...], v_ref[...], (((1,), (1,)), ((), ())),
        preferred_element_type=jnp.float32)
    lo = lax.bitcast_convert_type(s[:, : N // 2].astype(jnp.bfloat16)
                                  .astype(jnp.float32), jnp.int32)
    hi = lax.bitcast_convert_type(s[:, N // 2:].astype(jnp.bfloat16)
                                  .astype(jnp.float32), jnp.int32)
    s_ref[...] = jnp.bitwise_or(
        lax.shift_right_logical(lo, 16),
        jnp.bitwise_and(hi, jnp.int32(-65536)))


def _scores(w, values):
    return pl.pallas_call(
        _scores_body,
        grid=(N // BLKS,),
        in_specs=[
            pl.BlockSpec((BLKS, D), lambda i: (i, 0)),
            pl.BlockSpec((N, D), lambda i: (0, 0)),
        ],
        out_specs=pl.BlockSpec((BLKS, N // 2), lambda i: (i, 0)),
        out_shape=jax.ShapeDtypeStruct((N, N // 2), jnp.int32),
    )(w, values)


CH = 8
NCHUNK = ROWS_PER_W // CH


def _gather_body(scores_hbm, idx_hbm, out_hbm,
                 scr_v, idx_v, out_v, sem_in, sem_out):
    wid = lax.axis_index("s") * 2 + lax.axis_index("c")
    base = wid * ROWS_PER_W

    def start_in(c, b):
        rows = pl.ds(base + c * CH, CH)
        d1 = pltpu.async_copy(scores_hbm.at[rows], scr_v.at[b], sem_in.at[b])
        d2 = pltpu.async_copy(idx_hbm.at[rows], idx_v.at[b], sem_in.at[b])
        return d1, d2

    in_flight = {0: start_in(0, 0)}
    out_flight = {}
    for c in range(NCHUNK):
        b = c % 2
        if c + 1 < NCHUNK:
            in_flight[c + 1] = start_in(c + 1, (c + 1) % 2)
        for dsc in in_flight.pop(c):
            dsc.wait()
        if c - 2 in out_flight:
            out_flight.pop(c - 2).wait()
        for r in range(CH):
            rv = jnp.full((L,), r, jnp.int32)

            def gat(j, _, r=r, rv=rv, b=b):
                ivw = idx_v[b, r, pl.ds(j * L, L)]

                def one(iv):
                    w32 = plsc.load_gather(
                        scr_v.at[b], [rv, jnp.bitwise_and(iv, N // 2 - 1)])
                    sh = jnp.bitwise_and(lax.shift_right_logical(iv, 6), 16)
                    return lax.shift_left(lax.shift_right_logical(w32, sh), 16)

                blo = one(jnp.bitwise_and(ivw, N - 1))
                bhi = one(lax.shift_right_logical(ivw, 16))
                out_v[b, r, pl.ds(j * L, L)] = jnp.bitwise_or(
                    lax.shift_right_logical(blo, 16), bhi)
                return 0

            lax.fori_loop(0, D // (2 * L), gat, 0, unroll=2)
        out_flight[c] = pltpu.async_copy(
            out_v.at[b], out_hbm.at[pl.ds(base + c * CH, CH)], sem_out.at[b])
    for c in list(out_flight):
        out_flight.pop(c).wait()


def _sc_gather(scores, idx):
    mesh = plsc.VectorSubcoreMesh(core_axis_name="c", subcore_axis_name="s")
    f = pl.kernel(
        _gather_body,
        out_type=jax.ShapeDtypeStruct((N, D // 2), jnp.int32),
        mesh=mesh,
        compiler_params=pltpu.CompilerParams(needs_layout_passes=False),
        scratch_types=[
            pltpu.VMEM((2, CH, N // 2), jnp.int32),
            pltpu.VMEM((2, CH, D // 2), jnp.int32),
            pltpu.VMEM((2, CH, D // 2), jnp.int32),
            pltpu.SemaphoreType.DMA((2,)),
            pltpu.SemaphoreType.DMA((2,)),
        ],
    )
    return f(scores, idx)


def _out_body(vs_ref, wo_ref, bo_ref, o_ref):
    w32 = vs_ref[...]
    lo = lax.bitcast_convert_type(lax.shift_left(w32, 16), jnp.float32)
    hi = lax.bitcast_convert_type(
        jnp.bitwise_and(w32, jnp.int32(-65536)), jnp.float32)
    vs = jnp.concatenate([lo, hi], axis=1).astype(jnp.bfloat16)
    o_ref[...] = lax.dot_general(
        vs, wo_ref[...].astype(jnp.bfloat16),
        (((1,), (1,)), ((), ())),
        preferred_element_type=jnp.float32) + bo_ref[...]


def _out(vs, Wo, bo):
    return pl.pallas_call(
        _out_body,
        grid=(N // BLK1,),
        in_specs=[
            pl.BlockSpec((BLK1, D // 2), lambda i: (i, 0)),
            pl.BlockSpec((D, D), lambda i: (0, 0)),
            pl.BlockSpec((1, D), lambda i: (0, 0)),
        ],
        out_specs=pl.BlockSpec((BLK1, D), lambda i: (i, 0)),
        out_shape=jax.ShapeDtypeStruct((N, D), jnp.float32),
    )(vs, Wo, bo.reshape(1, D))


def kernel(x, Ws, bs, Ww, bw, Wv, bv, Wo, bo):
    idx, w, values = _stage1(x, Ws, bs, Ww, bw, Wv, bv)
    scores_w = _scores(w, values)
    vs = _sc_gather(scores_w, idx)
    return _out(vs, Wo, bo)


# speedup vs baseline: 1.1430x; 1.0733x over previous
import functools

import jax
import jax.numpy as jnp
from jax import lax
from jax.experimental import pallas as pl
from jax.experimental.pallas import tpu as pltpu
from jax.experimental.pallas import tpu_sc as plsc

N = 2048
D = 1024
BLK1 = 256
BLKS = 512
NW = 32
ROWS_PER_W = N // NW
L = 16


def _stage1_body(x_ref, ws_ref, bs_ref, ww_ref, bw_ref, wv_ref, bv_ref,
                 idx_ref, w_ref, v_ref):
    x = x_ref[...]
    dims = (((1,), (1,)), ((), ()))
    ls = lax.dot_general(x, ws_ref[...], dims,
                         preferred_element_type=jnp.float32) + bs_ref[...]
    m = jnp.max(ls, axis=-1, keepdims=True)
    e = jnp.exp(ls - m)
    p = e / jnp.sum(e, axis=-1, keepdims=True)
    idx = jnp.minimum((p * N).astype(jnp.int32), N - 1)
    idx_ref[...] = jnp.bitwise_or(idx[:, : D // 2],
                                  lax.shift_left(idx[:, D // 2:], 16))

    xb = x.astype(jnp.bfloat16)
    lw = lax.dot_general(xb, ww_ref[...].astype(jnp.bfloat16), dims,
                         preferred_element_type=jnp.float32) + bw_ref[...]
    mw = jnp.max(lw, axis=-1, keepdims=True)
    ew = jnp.exp(lw - mw)
    w_ref[...] = (ew / jnp.sum(ew, axis=-1, keepdims=True)).astype(jnp.bfloat16)

    v_ref[...] = (lax.dot_general(xb, wv_ref[...].astype(jnp.bfloat16), dims,
                                  preferred_element_type=jnp.float32)
                  + bv_ref[...]).astype(jnp.bfloat16)


def _stage1(x, Ws, bs, Ww, bw, Wv, bv):
    full = pl.BlockSpec((D, D), lambda i: (0, 0))
    bias = pl.BlockSpec((1, D), lambda i: (0, 0))
    rows = pl.BlockSpec((BLK1, D), lambda i: (i, 0))
    rows_h = pl.BlockSpec((BLK1, D // 2), lambda i: (i, 0))
    return pl.pallas_call(
        _stage1_body,
        grid=(N // BLK1,),
        in_specs=[rows, full, bias, full, bias, full, bias],
        out_specs=[rows_h, rows, rows],
        out_shape=[
            jax.ShapeDtypeStruct((N, D // 2), jnp.int32),
            jax.ShapeDtypeStruct((N, D), jnp.bfloat16),
            jax.ShapeDtypeStruct((N, D), jnp.bfloat16),
        ],
    )(x, Ws, bs.reshape(1, D), Ww, bw.reshape(1, D), Wv, bv.reshape(1, D))


def _scores_body(w_ref, v_ref, s_ref):
    s = lax.dot_general(
        w_ref[...], v_ref[...], (((1,), (1,)), ((), ())),
        preferred_element_type=jnp.float32)
    lo = lax.bitcast_convert_type(s[:, : N // 2].astype(jnp.bfloat16)
                                  .astype(jnp.float32), jnp.int32)
    hi = lax.bitcast_convert_type(s[:, N // 2:].astype(jnp.bfloat16)
                                  .astype(jnp.float32), jnp.int32)
    s_ref[...] = jnp.bitwise_or(
        lax.shift_right_logical(lo, 16),
        jnp.bitwise_and(hi, jnp.int32(-65536)))


def _scores(w, values):
    return pl.pallas_call(
        _scores_body,
        grid=(N // BLKS,),
        in_specs=[
            pl.BlockSpec((BLKS, D), lambda i: (i, 0)),
            pl.BlockSpec((N, D), lambda i: (0, 0)),
        ],
        out_specs=pl.BlockSpec((BLKS, N // 2), lambda i: (i, 0)),
        out_shape=jax.ShapeDtypeStruct((N, N // 2), jnp.int32),
    )(w, values)


CH = 8
NCHUNK = ROWS_PER_W // CH


def _gather_body(scores_hbm, idx_hbm, out_hbm,
                 scr_v, idx_v, out_v, sem_in, sem_out):
    wid = lax.axis_index("s") * 2 + lax.axis_index("c")
    base = wid * ROWS_PER_W

    def start_in(c, b):
        rows = pl.ds(base + c * CH, CH)
        d1 = pltpu.async_copy(scores_hbm.at[rows], scr_v.at[b], sem_in.at[b])
        d2 = pltpu.async_copy(idx_hbm.at[rows], idx_v.at[b], sem_in.at[b])
        return d1, d2

    in_flight = {0: start_in(0, 0)}
    out_flight = {}
    for c in range(NCHUNK):
        b = c % 2
        if c + 1 < NCHUNK:
            in_flight[c + 1] = start_in(c + 1, (c + 1) % 2)
        for dsc in in_flight.pop(c):
            dsc.wait()
        if c - 2 in out_flight:
            out_flight.pop(c - 2).wait()
        for r in range(CH):
            rv = jnp.full((L,), r, jnp.int32)

            def gat(j, _, r=r, rv=rv, b=b):
                ivw = idx_v[b, r, pl.ds(j * L, L)]

                def one(iv):
                    w32 = plsc.load_gather(
                        scr_v.at[b], [rv, jnp.bitwise_and(iv, N // 2 - 1)])
                    sh = jnp.bitwise_and(lax.shift_right_logical(iv, 6), 16)
                    return lax.shift_left(lax.shift_right_logical(w32, sh), 16)

                blo = one(jnp.bitwise_and(ivw, N - 1))
                bhi = one(lax.shift_right_logical(ivw, 16))
                out_v[b, r, pl.ds(j * L, L)] = jnp.bitwise_or(
                    lax.shift_right_logical(blo, 16), bhi)
                return 0

            lax.fori_loop(0, D // (2 * L), gat, 0)
        out_flight[c] = pltpu.async_copy(
            out_v.at[b], out_hbm.at[pl.ds(base + c * CH, CH)], sem_out.at[b])
    for c in list(out_flight):
        out_flight.pop(c).wait()


def _sc_gather(scores, idx):
    mesh = plsc.VectorSubcoreMesh(core_axis_name="c", subcore_axis_name="s")
    f = pl.kernel(
        _gather_body,
        out_type=jax.ShapeDtypeStruct((N, D // 2), jnp.int32),
        mesh=mesh,
        compiler_params=pltpu.CompilerParams(needs_layout_passes=False),
        scratch_types=[
            pltpu.VMEM((2, CH, N // 2), jnp.int32),
            pltpu.VMEM((2, CH, D // 2), jnp.int32),
            pltpu.VMEM((2, CH, D // 2), jnp.int32),
            pltpu.SemaphoreType.DMA((2,)),
            pltpu.SemaphoreType.DMA((2,)),
        ],
    )
    return f(scores, idx)


def _out_body(vs_ref, wo_ref, bo_ref, o_ref):
    w32 = vs_ref[...]
    lo = lax.bitcast_convert_type(lax.shift_left(w32, 16), jnp.float32)
    hi = lax.bitcast_convert_type(
        jnp.bitwise_and(w32, jnp.int32(-65536)), jnp.float32)
    vs = jnp.concatenate([lo, hi], axis=1).astype(jnp.bfloat16)
    o_ref[...] = lax.dot_general(
        vs, wo_ref[...].astype(jnp.bfloat16),
        (((1,), (1,)), ((), ())),
        preferred_element_type=jnp.float32) + bo_ref[...]


def _out(vs, Wo, bo):
    return pl.pallas_call(
        _out_body,
        grid=(N // BLK1,),
        in_specs=[
            pl.BlockSpec((BLK1, D // 2), lambda i: (i, 0)),
            pl.BlockSpec((D, D), lambda i: (0, 0)),
            pl.BlockSpec((1, D), lambda i: (0, 0)),
        ],
        out_specs=pl.BlockSpec((BLK1, D), lambda i: (i, 0)),
        out_shape=jax.ShapeDtypeStruct((N, D), jnp.float32),
    )(vs, Wo, bo.reshape(1, D))


def kernel(x, Ws, bs, Ww, bw, Wv, bv, Wo, bo):
    idx, w, values = _stage1(x, Ws, bs, Ww, bw, Wv, bv)
    scores_w = _scores(w, values)
    vs = _sc_gather(scores_w, idx)
    return _out(vs, Wo, bo)


# speedup vs baseline: 1.1874x; 1.0389x over previous
import functools

import jax
import jax.numpy as jnp
from jax import lax
from jax.experimental import pallas as pl
from jax.experimental.pallas import tpu as pltpu
from jax.experimental.pallas import tpu_sc as plsc

N = 2048
D = 1024
BLK1 = 512
BLKS = 512
NW = 32
ROWS_PER_W = N // NW
L = 16


def _stage1_body(x_ref, ws_ref, bs_ref, ww_ref, bw_ref, wv_ref, bv_ref,
                 idx_ref, w_ref, v_ref):
    x = x_ref[...]
    dims = (((1,), (1,)), ((), ()))
    ls = lax.dot_general(x, ws_ref[...], dims,
                         preferred_element_type=jnp.float32) + bs_ref[...]
    m = jnp.max(ls, axis=-1, keepdims=True)
    e = jnp.exp(ls - m)
    p = e / jnp.sum(e, axis=-1, keepdims=True)
    idx = jnp.minimum((p * N).astype(jnp.int32), N - 1)
    idx_ref[...] = jnp.bitwise_or(idx[:, : D // 2],
                                  lax.shift_left(idx[:, D // 2:], 16))

    xb = x.astype(jnp.bfloat16)
    lw = lax.dot_general(xb, ww_ref[...].astype(jnp.bfloat16), dims,
                         preferred_element_type=jnp.float32) + bw_ref[...]
    mw = jnp.max(lw, axis=-1, keepdims=True)
    ew = jnp.exp(lw - mw)
    w_ref[...] = (ew / jnp.sum(ew, axis=-1, keepdims=True)).astype(jnp.bfloat16)

    v_ref[...] = (lax.dot_general(xb, wv_ref[...].astype(jnp.bfloat16), dims,
                                  preferred_element_type=jnp.float32)
                  + bv_ref[...]).astype(jnp.bfloat16)


def _stage1(x, Ws, bs, Ww, bw, Wv, bv):
    full = pl.BlockSpec((D, D), lambda i: (0, 0))
    bias = pl.BlockSpec((1, D), lambda i: (0, 0))
    rows = pl.BlockSpec((BLK1, D), lambda i: (i, 0))
    rows_h = pl.BlockSpec((BLK1, D // 2), lambda i: (i, 0))
    return pl.pallas_call(
        _stage1_body,
        grid=(N // BLK1,),
        in_specs=[rows, full, bias, full, bias, full, bias],
        out_specs=[rows_h, rows, rows],
        out_shape=[
            jax.ShapeDtypeStruct((N, D // 2), jnp.int32),
            jax.ShapeDtypeStruct((N, D), jnp.bfloat16),
            jax.ShapeDtypeStruct((N, D), jnp.bfloat16),
        ],
    )(x, Ws, bs.reshape(1, D), Ww, bw.reshape(1, D), Wv, bv.reshape(1, D))


def _scores_body(w_ref, v_ref, s_ref):
    s = lax.dot_general(
        w_ref[...], v_ref[...], (((1,), (1,)), ((), ())),
        preferred_element_type=jnp.float32)
    lo = lax.bitcast_convert_type(s[:, : N // 2].astype(jnp.bfloat16)
                                  .astype(jnp.float32), jnp.int32)
    hi = lax.bitcast_convert_type(s[:, N // 2:].astype(jnp.bfloat16)
                                  .astype(jnp.float32), jnp.int32)
    s_ref[...] = jnp.bitwise_or(
        lax.shift_right_logical(lo, 16),
        jnp.bitwise_and(hi, jnp.int32(-65536)))


def _scores(w, values):
    return pl.pallas_call(
        _scores_body,
        grid=(N // BLKS,),
        in_specs=[
            pl.BlockSpec((BLKS, D), lambda i: (i, 0)),
            pl.BlockSpec((N, D), lambda i: (0, 0)),
        ],
        out_specs=pl.BlockSpec((BLKS, N // 2), lambda i: (i, 0)),
        out_shape=jax.ShapeDtypeStruct((N, N // 2), jnp.int32),
    )(w, values)


CH = 8
NCHUNK = ROWS_PER_W // CH


def _gather_body(scores_hbm, idx_hbm, out_hbm,
                 scr_v, idx_v, out_v, sem_in, sem_out):
    wid = lax.axis_index("s") * 2 + lax.axis_index("c")
    base = wid * ROWS_PER_W

    def start_in(c, b):
        rows = pl.ds(base + c * CH, CH)
        d1 = pltpu.async_copy(scores_hbm.at[rows], scr_v.at[b], sem_in.at[b])
        d2 = pltpu.async_copy(idx_hbm.at[rows], idx_v.at[b], sem_in.at[b])
        return d1, d2

    in_flight = {0: start_in(0, 0)}
    out_flight = {}
    for c in range(NCHUNK):
        b = c % 2
        if c + 1 < NCHUNK:
            in_flight[c + 1] = start_in(c + 1, (c + 1) % 2)
        for dsc in in_flight.pop(c):
            dsc.wait()
        if c - 2 in out_flight:
            out_flight.pop(c - 2).wait()
        for r in range(CH):
            rv = jnp.full((L,), r, jnp.int32)

            def gat(j, _, r=r, rv=rv, b=b):
                ivw = idx_v[b, r, pl.ds(j * L, L)]

                def one(iv):
                    w32 = plsc.load_gather(
                        scr_v.at[b], [rv, jnp.bitwise_and(iv, N // 2 - 1)])
                    sh = jnp.bitwise_and(lax.shift_right_logical(iv, 6), 16)
                    return lax.shift_left(lax.shift_right_logical(w32, sh), 16)

                blo = one(jnp.bitwise_and(ivw, N - 1))
                bhi = one(lax.shift_right_logical(ivw, 16))
                out_v[b, r, pl.ds(j * L, L)] = jnp.bitwise_or(
                    lax.shift_right_logical(blo, 16), bhi)
                return 0

            lax.fori_loop(0, D // (2 * L), gat, 0)
        out_flight[c] = pltpu.async_copy(
            out_v.at[b], out_hbm.at[pl.ds(base + c * CH, CH)], sem_out.at[b])
    for c in list(out_flight):
        out_flight.pop(c).wait()


def _sc_gather(scores, idx):
    mesh = plsc.VectorSubcoreMesh(core_axis_name="c", subcore_axis_name="s")
    f = pl.kernel(
        _gather_body,
        out_type=jax.ShapeDtypeStruct((N, D // 2), jnp.int32),
        mesh=mesh,
        compiler_params=pltpu.CompilerParams(needs_layout_passes=False),
        scratch_types=[
            pltpu.VMEM((2, CH, N // 2), jnp.int32),
            pltpu.VMEM((2, CH, D // 2), jnp.int32),
            pltpu.VMEM((2, CH, D // 2), jnp.int32),
            pltpu.SemaphoreType.DMA((2,)),
            pltpu.SemaphoreType.DMA((2,)),
        ],
    )
    return f(scores, idx)


def _out_body(vs_ref, wo_ref, bo_ref, o_ref):
    w32 = vs_ref[...]
    lo = lax.bitcast_convert_type(lax.shift_left(w32, 16), jnp.float32)
    hi = lax.bitcast_convert_type(
        jnp.bitwise_and(w32, jnp.int32(-65536)), jnp.float32)
    vs = jnp.concatenate([lo, hi], axis=1).astype(jnp.bfloat16)
    o_ref[...] = lax.dot_general(
        vs, wo_ref[...].astype(jnp.bfloat16),
        (((1,), (1,)), ((), ())),
        preferred_element_type=jnp.float32) + bo_ref[...]


def _out(vs, Wo, bo):
    return pl.pallas_call(
        _out_body,
        grid=(N // BLK1,),
        in_specs=[
            pl.BlockSpec((BLK1, D // 2), lambda i: (i, 0)),
            pl.BlockSpec((D, D), lambda i: (0, 0)),
            pl.BlockSpec((1, D), lambda i: (0, 0)),
        ],
        out_specs=pl.BlockSpec((BLK1, D), lambda i: (i, 0)),
        out_shape=jax.ShapeDtypeStruct((N, D), jnp.float32),
    )(vs, Wo, bo.reshape(1, D))


def kernel(x, Ws, bs, Ww, bw, Wv, bv, Wo, bo):
    idx, w, values = _stage1(x, Ws, bs, Ww, bw, Wv, bv)
    scores_w = _scores(w, values)
    vs = _sc_gather(scores_w, idx)
    return _out(vs, Wo, bo)


# speedup vs baseline: 1.1925x; 1.0043x over previous
import functools

import jax
import jax.numpy as jnp
from jax import lax
from jax.experimental import pallas as pl
from jax.experimental.pallas import tpu as pltpu
from jax.experimental.pallas import tpu_sc as plsc

N = 2048
D = 1024
BLK1 = 1024
BLKS = 512
NW = 32
ROWS_PER_W = N // NW
L = 16


def _stage1_body(x_ref, ws_ref, bs_ref, ww_ref, bw_ref, wv_ref, bv_ref,
                 idx_ref, w_ref, v_ref):
    x = x_ref[...]
    dims = (((1,), (1,)), ((), ()))
    ls = lax.dot_general(x, ws_ref[...], dims,
                         preferred_element_type=jnp.float32) + bs_ref[...]
    m = jnp.max(ls, axis=-1, keepdims=True)
    e = jnp.exp(ls - m)
    p = e / jnp.sum(e, axis=-1, keepdims=True)
    idx = jnp.minimum((p * N).astype(jnp.int32), N - 1)
    idx_ref[...] = jnp.bitwise_or(idx[:, : D // 2],
                                  lax.shift_left(idx[:, D // 2:], 16))

    xb = x.astype(jnp.bfloat16)
    lw = lax.dot_general(xb, ww_ref[...].astype(jnp.bfloat16), dims,
                         preferred_element_type=jnp.float32) + bw_ref[...]
    mw = jnp.max(lw, axis=-1, keepdims=True)
    ew = jnp.exp(lw - mw)
    w_ref[...] = (ew / jnp.sum(ew, axis=-1, keepdims=True)).astype(jnp.bfloat16)

    v_ref[...] = (lax.dot_general(xb, wv_ref[...].astype(jnp.bfloat16), dims,
                                  preferred_element_type=jnp.float32)
                  + bv_ref[...]).astype(jnp.bfloat16)


def _stage1(x, Ws, bs, Ww, bw, Wv, bv):
    full = pl.BlockSpec((D, D), lambda i: (0, 0))
    bias = pl.BlockSpec((1, D), lambda i: (0, 0))
    rows = pl.BlockSpec((BLK1, D), lambda i: (i, 0))
    rows_h = pl.BlockSpec((BLK1, D // 2), lambda i: (i, 0))
    return pl.pallas_call(
        _stage1_body,
        grid=(N // BLK1,),
        in_specs=[rows, full, bias, full, bias, full, bias],
        out_specs=[rows_h, rows, rows],
        out_shape=[
            jax.ShapeDtypeStruct((N, D // 2), jnp.int32),
            jax.ShapeDtypeStruct((N, D), jnp.bfloat16),
            jax.ShapeDtypeStruct((N, D), jnp.bfloat16),
        ],
    )(x, Ws, bs.reshape(1, D), Ww, bw.reshape(1, D), Wv, bv.reshape(1, D))


def _scores_body(w_ref, v_ref, s_ref):
    s = lax.dot_general(
        w_ref[...], v_ref[...], (((1,), (1,)), ((), ())),
        preferred_element_type=jnp.float32)
    lo = lax.bitcast_convert_type(s[:, : N // 2].astype(jnp.bfloat16)
                                  .astype(jnp.float32), jnp.int32)
    hi = lax.bitcast_convert_type(s[:, N // 2:].astype(jnp.bfloat16)
                                  .astype(jnp.float32), jnp.int32)
    s_ref[...] = jnp.bitwise_or(
        lax.shift_right_logical(lo, 16),
        jnp.bitwise_and(hi, jnp.int32(-65536)))


def _scores(w, values):
    return pl.pallas_call(
        _scores_body,
        grid=(N // BLKS,),
        in_specs=[
            pl.BlockSpec((BLKS, D), lambda i: (i, 0)),
            pl.BlockSpec((N, D), lambda i: (0, 0)),
        ],
        out_specs=pl.BlockSpec((BLKS, N // 2), lambda i: (i, 0)),
        out_shape=jax.ShapeDtypeStruct((N, N // 2), jnp.int32),
    )(w, values)


CH = 8
NCHUNK = ROWS_PER_W // CH


def _gather_body(scores_hbm, idx_hbm, out_hbm,
                 scr_v, idx_v, out_v, sem_in, sem_out):
    wid = lax.axis_index("s") * 2 + lax.axis_index("c")
    base = wid * ROWS_PER_W

    def start_in(c, b):
        rows = pl.ds(base + c * CH, CH)
        d1 = pltpu.async_copy(scores_hbm.at[rows], scr_v.at[b], sem_in.at[b])
        d2 = pltpu.async_copy(idx_hbm.at[rows], idx_v.at[b], sem_in.at[b])
        return d1, d2

    in_flight = {0: start_in(0, 0)}
    out_flight = {}
    for c in range(NCHUNK):
        b = c % 2
        if c + 1 < NCHUNK:
            in_flight[c + 1] = start_in(c + 1, (c + 1) % 2)
        for dsc in in_flight.pop(c):
            dsc.wait()
        if c - 2 in out_flight:
            out_flight.pop(c - 2).wait()
        for r in range(CH):
            rv = jnp.full((L,), r, jnp.int32)

            def gat(j, _, r=r, rv=rv, b=b):
                ivw = idx_v[b, r, pl.ds(j * L, L)]

                def one(iv):
                    w32 = plsc.load_gather(
                        scr_v.at[b], [rv, jnp.bitwise_and(iv, N // 2 - 1)])
                    sh = jnp.bitwise_and(lax.shift_right_logical(iv, 6), 16)
                    return lax.shift_left(lax.shift_right_logical(w32, sh), 16)

                blo = one(jnp.bitwise_and(ivw, N - 1))
                bhi = one(lax.shift_right_logical(ivw, 16))
                out_v[b, r, pl.ds(j * L, L)] = jnp.bitwise_or(
                    lax.shift_right_logical(blo, 16), bhi)
                return 0

            lax.fori_loop(0, D // (2 * L), gat, 0)
        out_flight[c] = pltpu.async_copy(
            out_v.at[b], out_hbm.at[pl.ds(base + c * CH, CH)], sem_out.at[b])
    for c in list(out_flight):
        out_flight.pop(c).wait()


def _sc_gather(scores, idx):
    mesh = plsc.VectorSubcoreMesh(core_axis_name="c", subcore_axis_name="s")
    f = pl.kernel(
        _gather_body,
        out_type=jax.ShapeDtypeStruct((N, D // 2), jnp.int32),
        mesh=mesh,
        compiler_params=pltpu.CompilerParams(needs_layout_passes=False),
        scratch_types=[
            pltpu.VMEM((2, CH, N // 2), jnp.int32),
            pltpu.VMEM((2, CH, D // 2), jnp.int32),
            pltpu.VMEM((2, CH, D // 2), jnp.int32),
            pltpu.SemaphoreType.DMA((2,)),
            pltpu.SemaphoreType.DMA((2,)),
        ],
    )
    return f(scores, idx)


def _out_body(vs_ref, wo_ref, bo_ref, o_ref):
    w32 = vs_ref[...]
    lo = lax.bitcast_convert_type(lax.shift_left(w32, 16), jnp.float32)
    hi = lax.bitcast_convert_type(
        jnp.bitwise_and(w32, jnp.int32(-65536)), jnp.float32)
    vs = jnp.concatenate([lo, hi], axis=1).astype(jnp.bfloat16)
    o_ref[...] = lax.dot_general(
        vs, wo_ref[...].astype(jnp.bfloat16),
        (((1,), (1,)), ((), ())),
        preferred_element_type=jnp.float32) + bo_ref[...]


def _out(vs, Wo, bo):
    return pl.pallas_call(
        _out_body,
        grid=(N // BLK1,),
        in_specs=[
            pl.BlockSpec((BLK1, D // 2), lambda i: (i, 0)),
            pl.BlockSpec((D, D), lambda i: (0, 0)),
            pl.BlockSpec((1, D), lambda i: (0, 0)),
        ],
        out_specs=pl.BlockSpec((BLK1, D), lambda i: (i, 0)),
        out_shape=jax.ShapeDtypeStruct((N, D), jnp.float32),
    )(vs, Wo, bo.reshape(1, D))


def kernel(x, Ws, bs, Ww, bw, Wv, bv, Wo, bo):
    idx, w, values = _stage1(x, Ws, bs, Ww, bw, Wv, bv)
    scores_w = _scores(w, values)
    vs = _sc_gather(scores_w, idx)
    return _out(vs, Wo, bo)
